# baseline (speedup 1.0000x reference)
import functools

import jax
import jax.numpy as jnp
from jax import lax
from jax.experimental import pallas as pl
from jax.experimental.pallas import tpu as pltpu
from jax.experimental.pallas import tpu_sc as plsc

N_CAT = 1000
L = 16
NC = 2
NS = 16
R = 512
RT = R // NS
NSLOT = 2
ZCHUNK = 8000


def _one_hot_sc(x_flat, n_rows):
    rows_per_core = n_rows // NC
    n_blocks = rows_per_core // R
    b_per_w = n_rows // (NC * NS)
    mesh = plsc.VectorSubcoreMesh(core_axis_name="c", subcore_axis_name="s")

    @functools.partial(
        pl.kernel,
        out_type=jax.ShapeDtypeStruct((n_rows * N_CAT,), jnp.float32),
        mesh=mesh,
        scratch_types=[
            pltpu.VMEM_SHARED((NSLOT * R * N_CAT,), jnp.float32),
            pltpu.VMEM((b_per_w,), jnp.int32),
            pltpu.VMEM((RT,), jnp.int32),
            pltpu.VMEM((RT,), jnp.float32),
            pltpu.VMEM((RT,), jnp.float32),
            pltpu.VMEM((ZCHUNK,), jnp.float32),
            pltpu.SemaphoreType.DMA,
            [pltpu.SemaphoreType.DMA] * NSLOT,
        ],
        compiler_params=pltpu.CompilerParams(needs_layout_passes=False),
    )
    def body(x_hbm, out_hbm, spmem, idx_v, pos_v, ones_v, zvals_v, zbuf_v,
             ld_sem, drain_sems):
        c = lax.axis_index("c")
        s = lax.axis_index("s")
        core_base = c * rows_per_core

        zeros = jnp.zeros((L,), jnp.float32)
        ones = jnp.ones((L,), jnp.float32)
        lane = lax.iota(jnp.int32, L)

        for g in range(RT // L):
            ones_v[pl.ds(g * L, L)] = ones
            zvals_v[pl.ds(g * L, L)] = zeros

        def zb_body(i, _):
            zbuf_v[pl.ds(i * L, L)] = zeros
            return 0

        lax.fori_loop(0, ZCHUNK // L, zb_body, 0)

        def idx_dma(b):
            return pltpu.make_async_copy(
                x_hbm.at[pl.ds(core_base + b * R + s * RT, RT)],
                idx_v.at[pl.ds(b * RT, RT)],
                ld_sem,
            )

        def idx_fire(b, _):
            idx_dma(b).start()
            return 0

        def idx_drain(b, _):
            idx_dma(b).wait()
            return 0

        lax.fori_loop(0, n_blocks, idx_fire, 0)

        share = (NSLOT * R * N_CAT) // NS
        for j in range(share // ZCHUNK):
            pltpu.sync_copy(
                zbuf_v, spmem.at[pl.ds(s * share + j * ZCHUNK, ZCHUNK)]
            )

        lax.fori_loop(0, n_blocks, idx_drain, 0)
        plsc.subcore_barrier()

        def fill_pos(b, slot):
            for g in range(RT // L):
                cols = idx_v[pl.ds(b * RT + g * L, L)]
                row = s * RT + g * L + lane
                pos_v[pl.ds(g * L, L)] = (
                    slot * (R * N_CAT) + row * N_CAT + cols
                )

        def scatter(vals_v):
            pltpu.sync_copy(vals_v, spmem.at[pos_v])

        def drain(slot, b):
            return pltpu.make_async_copy(
                spmem.at[pl.ds(slot * R * N_CAT, R * N_CAT)],
                out_hbm.at[pl.ds((core_base + b * R) * N_CAT, R * N_CAT)],
                drain_sems[slot],
            )

        for slot in range(NSLOT):
            fill_pos(slot, slot)
            scatter(ones_v)
            plsc.subcore_barrier()

            @pl.when(s == 0)
            def _():
                drain(slot, slot).start()

        def group_body(g, _):
            for slot in range(NSLOT):
                b = g * NSLOT + slot

                @pl.when(s == 0)
                def _():
                    drain(slot, b - NSLOT).wait()

                plsc.subcore_barrier()
                fill_pos(b - NSLOT, slot)
                scatter(zvals_v)
                fill_pos(b, slot)
                scatter(ones_v)
                plsc.subcore_barrier()

                @pl.when(s == 0)
                def _():
                    drain(slot, b).start()

            return 0

        lax.fori_loop(1, n_blocks // NSLOT, group_body, 0)

        @pl.when(s == 0)
        def _():
            for slot in range(NSLOT):
                drain(slot, n_blocks - NSLOT + slot).wait()

    return body(x_flat)


def kernel(x, eye):
    n_rows = x.shape[0] * x.shape[1]
    x_flat = x.reshape(n_rows).astype(jnp.int32)
    out_flat = _one_hot_sc(x_flat, n_rows)
    return out_flat.reshape(x.shape[0], x.shape[1], N_CAT)
